# Initial kernel scaffold; baseline (speedup 1.0000x reference)
import jax
import jax.numpy as jnp
from jax.experimental import pallas as pl


def kernel(x, edge_index, W1, a_src1, a_dst1, b1, W2, a_src2, a_dst2, b2):
    raise NotImplementedError("write your pallas kernel here")


# speedup vs baseline: 45.9316x; 45.9316x over previous
import functools

import jax
import jax.numpy as jnp
from jax import lax
from jax.experimental import pallas as pl
from jax.experimental.pallas import tpu as pltpu
from jax.experimental.pallas import tpu_sc as plsc

N = 10000
D_IN = 128
H1, C1 = 16, 8
H2, C2 = 1, 64

NC, NS, L = 2, 16, 16
NW = NC * NS
K = 128

N_PAD = 10240
E_TOT = 320000 + N
CHUNKS = -(-E_TOT // (NW * K))
E_PAD = NW * K * CHUNKS
PER_TILE = E_PAD // NW
ROWS_PER_TILE = N_PAD // NS


def _make_edge_kernel(df):
    nv = df // L
    mesh = plsc.VectorSubcoreMesh(
        core_axis_name="c", subcore_axis_name="s",
        num_cores=NC, num_subcores=NS)

    @functools.partial(
        pl.kernel,
        out_type=(
            jax.ShapeDtypeStruct((NC * N_PAD, df), jnp.float32),
            jax.ShapeDtypeStruct((NC * N_PAD, L), jnp.float32),
        ),
        mesh=mesh,
        compiler_params=pltpu.CompilerParams(use_tc_tiling_on_sc=False),
        scratch_types=[
            pltpu.VMEM((K,), jnp.int32),
            pltpu.VMEM((K,), jnp.int32),
            pltpu.VMEM((K, L), jnp.float32),
            pltpu.VMEM((K, L), jnp.float32),
            pltpu.VMEM((K, df), jnp.float32),
            pltpu.VMEM_SHARED((N_PAD, df), jnp.float32),
            pltpu.VMEM_SHARED((N_PAD, L), jnp.float32),
            pltpu.SemaphoreType.DMA,
            pltpu.SemaphoreType.DMA,
            pltpu.SemaphoreType.DMA,
        ],
    )
    def edge_kernel(src_hbm, dst_hbm, asrc_hbm, adst_hbm, table_hbm,
                    outn_hbm, outd_hbm,
                    src_v, dst_v, as_v, ad_v, feat_v, acc_n, acc_d,
                    sem0, sem1, sem2):
        cid = lax.axis_index("c")
        sid = lax.axis_index("s")
        wid = sid * NC + cid

        zero = jnp.zeros((L,), jnp.float32)

        def zero_body(k, _):
            for j in range(nv):
                feat_v[k, pl.ds(L * j, L)] = zero
            ad_v[k] = zero
            return 0

        lax.fori_loop(0, K, zero_body, 0)
        row0 = sid * ROWS_PER_TILE
        for i in range(ROWS_PER_TILE // K):
            pltpu.sync_copy(feat_v, acc_n.at[pl.ds(row0 + i * K, K)])
            pltpu.sync_copy(ad_v, acc_d.at[pl.ds(row0 + i * K, K)])
        plsc.subcore_barrier()

        def chunk_body(c, _):
            base = wid * PER_TILE + c * K
            pltpu.sync_copy(src_hbm.at[pl.ds(base, K)], src_v)
            pltpu.sync_copy(dst_hbm.at[pl.ds(base, K)], dst_v)
            cp0 = pltpu.async_copy(asrc_hbm.at[src_v], as_v, sem0)
            cp1 = pltpu.async_copy(adst_hbm.at[dst_v], ad_v, sem1)
            cp2 = pltpu.async_copy(table_hbm.at[src_v], feat_v, sem2)
            cp0.wait()
            cp1.wait()
            cp2.wait()

            def edge_body(k, _):
                e = as_v[k] + ad_v[k]
                e = jnp.maximum(e, e * 0.2)
                ex = jnp.exp(e)
                ad_v[k] = ex
                for j in range(nv):
                    feat_v[k, pl.ds(L * j, L)] = feat_v[k, pl.ds(L * j, L)] * ex
                return 0

            lax.fori_loop(0, K, edge_body, 0)
            pltpu.sync_copy(feat_v, acc_n.at[dst_v], add=True)
            pltpu.sync_copy(ad_v, acc_d.at[dst_v], add=True)
            return 0

        lax.fori_loop(0, CHUNKS, chunk_body, 0)
        plsc.subcore_barrier()

        pltpu.sync_copy(acc_n.at[pl.ds(row0, ROWS_PER_TILE)],
                        outn_hbm.at[pl.ds(cid * N_PAD + row0, ROWS_PER_TILE)])
        pltpu.sync_copy(acc_d.at[pl.ds(row0, ROWS_PER_TILE)],
                        outd_hbm.at[pl.ds(cid * N_PAD + row0, ROWS_PER_TILE)])

    return edge_kernel


_BLK = 512
_GRID = N_PAD // _BLK


def _stage0(x_pad, w1p, a_s, a_d):
    def body(x_ref, w_ref, s_ref, d_ref, t_out, s_out, d_out):
        xb = x_ref[...]
        t_out[...] = jnp.dot(xb, w_ref[...], preferred_element_type=jnp.float32)
        s_out[...] = jnp.dot(xb, s_ref[...], preferred_element_type=jnp.float32)
        d_out[...] = jnp.dot(xb, d_ref[...], preferred_element_type=jnp.float32)

    return pl.pallas_call(
        body,
        grid=(_GRID,),
        in_specs=[
            pl.BlockSpec((_BLK, D_IN), lambda i: (i, 0)),
            pl.BlockSpec((D_IN, H1 * C1), lambda i: (0, 0)),
            pl.BlockSpec((D_IN, H1), lambda i: (0, 0)),
            pl.BlockSpec((D_IN, H1), lambda i: (0, 0)),
        ],
        out_specs=[
            pl.BlockSpec((_BLK, H1 * C1), lambda i: (i, 0)),
            pl.BlockSpec((_BLK, H1), lambda i: (i, 0)),
            pl.BlockSpec((_BLK, H1), lambda i: (i, 0)),
        ],
        out_shape=[
            jax.ShapeDtypeStruct((N_PAD, H1 * C1), jnp.float32),
            jax.ShapeDtypeStruct((N_PAD, H1), jnp.float32),
            jax.ShapeDtypeStruct((N_PAD, H1), jnp.float32),
        ],
    )(x_pad, w1p, a_s, a_d)


def _stage1(accn, accd, w2p, a2, b1p):
    def body(n0_ref, n1_ref, d0_ref, d1_ref, w_ref, a2_ref, b_ref,
             t_out, s_out, d_out):
        num = n0_ref[...] + n1_ref[...]
        den = d0_ref[...] + d1_ref[...]
        denb = jnp.concatenate([den] * C1, axis=1)
        h = num / (denb + 1e-16) + b_ref[...]
        t_out[...] = jnp.dot(h, w_ref[...], preferred_element_type=jnp.float32)
        ysd = jnp.dot(h, a2_ref[...], preferred_element_type=jnp.float32)
        s_out[...] = jnp.broadcast_to(ysd[:, 0:1], (_BLK, L))
        d_out[...] = jnp.broadcast_to(ysd[:, 1:2], (_BLK, L))

    return pl.pallas_call(
        body,
        grid=(_GRID,),
        in_specs=[
            pl.BlockSpec((_BLK, H1 * C1), lambda i: (i, 0)),
            pl.BlockSpec((_BLK, H1 * C1), lambda i: (i + _GRID, 0)),
            pl.BlockSpec((_BLK, L), lambda i: (i, 0)),
            pl.BlockSpec((_BLK, L), lambda i: (i + _GRID, 0)),
            pl.BlockSpec((H1 * C1, C2), lambda i: (0, 0)),
            pl.BlockSpec((H1 * C1, 2), lambda i: (0, 0)),
            pl.BlockSpec((1, H1 * C1), lambda i: (0, 0)),
        ],
        out_specs=[
            pl.BlockSpec((_BLK, C2), lambda i: (i, 0)),
            pl.BlockSpec((_BLK, L), lambda i: (i, 0)),
            pl.BlockSpec((_BLK, L), lambda i: (i, 0)),
        ],
        out_shape=[
            jax.ShapeDtypeStruct((N_PAD, C2), jnp.float32),
            jax.ShapeDtypeStruct((N_PAD, L), jnp.float32),
            jax.ShapeDtypeStruct((N_PAD, L), jnp.float32),
        ],
    )(accn, accn, accd, accd, w2p, a2, b1p)


def _stage2(accn, accd, b2r):
    def body(n0_ref, n1_ref, d0_ref, d1_ref, b_ref, o_ref):
        num = n0_ref[...] + n1_ref[...]
        den = d0_ref[...] + d1_ref[...]
        o_ref[...] = jax.nn.sigmoid(
            num / (den[:, 0:1] + 1e-16) + b_ref[...])

    return pl.pallas_call(
        body,
        grid=(_GRID,),
        in_specs=[
            pl.BlockSpec((_BLK, C2), lambda i: (i, 0)),
            pl.BlockSpec((_BLK, C2), lambda i: (i + _GRID, 0)),
            pl.BlockSpec((_BLK, L), lambda i: (i, 0)),
            pl.BlockSpec((_BLK, L), lambda i: (i + _GRID, 0)),
            pl.BlockSpec((1, C2), lambda i: (0, 0)),
        ],
        out_specs=pl.BlockSpec((_BLK, C2), lambda i: (i, 0)),
        out_shape=jax.ShapeDtypeStruct((N_PAD, C2), jnp.float32),
    )(accn, accn, accd, accd, b2r)


def kernel(x, edge_index, W1, a_src1, a_dst1, b1, W2, a_src2, a_dst2, b2):
    loop = jnp.arange(N, dtype=jnp.int32)
    pad = jnp.full((E_PAD - E_TOT,), N, dtype=jnp.int32)
    src = jnp.concatenate([edge_index[0], loop, pad])
    dst = jnp.concatenate([edge_index[1], loop, pad])
    x_pad = jnp.pad(x, ((0, N_PAD - N), (0, 0)))

    w1r = W1.reshape(D_IN, H1, C1)
    w1p = w1r.transpose(0, 2, 1).reshape(D_IN, H1 * C1)
    a_s1 = jnp.einsum("ihc,hc->ih", w1r, a_src1)
    a_d1 = jnp.einsum("ihc,hc->ih", w1r, a_dst1)
    b1p = b1.reshape(H1, C1).T.reshape(1, H1 * C1)
    w2p = W2.reshape(H1, C1, C2).transpose(1, 0, 2).reshape(H1 * C1, C2)
    a2 = jnp.dot(w2p, jnp.stack([a_src2[0], a_dst2[0]], axis=1))
    b2r = b2.reshape(1, C2)

    table1, as1, ad1 = _stage0(x_pad, w1p, a_s1, a_d1)
    accn1, accd1 = _make_edge_kernel(H1 * C1)(src, dst, as1, ad1, table1)

    table2, as2, ad2 = _stage1(accn1, accd1, w2p, a2, b1p)
    accn2, accd2 = _make_edge_kernel(C2)(src, dst, as2, ad2, table2)

    out = _stage2(accn2, accd2, b2r)
    return out[:N]
